# baseline (speedup 1.0000x reference)
import jax
import jax.numpy as jnp
from jax.experimental import pallas as pl
from jax.experimental.pallas import tpu as pltpu

N_B, L_S = 128, 160
V_SZ, E_D = 30000, 512
H_E = 512
H_D = 1024
T_T = 128
BT = 8
G_T = L_S // BT
NH = N_B // 2


def _enc0_body(ids_ref, embed_ref, wx_ref, wh_ref, b_ref, out_ref,
               xg_ref, gin_ref, h_ref, c_ref, sem):
    d = pl.program_id(0)
    i = pl.program_id(1)
    cur = jax.lax.rem(i, 2)
    nxt = 1 - cur

    def issue_block(blk, slot):
        base = blk * (BT * N_B)

        def issue8(k8, carry):
            k = k8 * 8
            for u in range(8):
                tok = ids_ref[base + k + u]
                pltpu.make_async_copy(embed_ref.at[tok],
                                      xg_ref.at[slot, k + u], sem).start()
            return carry

        jax.lax.fori_loop(0, (BT * N_B) // 8, issue8, 0)

    @pl.when(i == 0)
    def _():
        h_ref[...] = jnp.zeros((N_B, H_E), jnp.float32)
        c_ref[...] = jnp.zeros((N_B, H_E), jnp.float32)
        issue_block(d * (G_T - 1), cur)
        pltpu.make_async_copy(xg_ref.at[cur], xg_ref.at[cur], sem).wait()

    @pl.when(i < G_T - 1)
    def _():
        ii = i + 1
        issue_block(ii + d * (G_T - 1 - 2 * ii), nxt)

    gin_ref[...] = jnp.dot(xg_ref[cur], wx_ref[0],
                           preferred_element_type=jnp.float32)

    for j in range(BT):
        row = j + d * (BT - 1 - 2 * j)
        base = pl.multiple_of(row * N_B, N_B)
        g = (gin_ref[pl.ds(base, N_B), :] + jnp.dot(
            h_ref[...], wh_ref[0],
            preferred_element_type=jnp.float32)) + b_ref[0]
        gi = _sigm(g[:, :H_E])
        gf = _sigm(g[:, H_E:2 * H_E])
        gg = jnp.tanh(g[:, 2 * H_E:3 * H_E])
        go = _sigm(g[:, 3 * H_E:])
        c = gf * c_ref[...] + gi * gg
        c_ref[...] = c
        h = go * jnp.tanh(c)
        h_ref[...] = h
        out_ref[row] = h

    @pl.when(i < G_T - 1)
    def _():
        pltpu.make_async_copy(xg_ref.at[nxt], xg_ref.at[nxt], sem).wait()


def _enc0_layer(ids_flat, embed, wx, wh, b):
    return pl.pallas_call(
        _enc0_body,
        grid=(2, G_T),
        in_specs=[
            pl.BlockSpec(memory_space=pltpu.SMEM),
            pl.BlockSpec(memory_space=pl.ANY),
            pl.BlockSpec((1, E_D, 4 * H_E), lambda d, i: (d, 0, 0)),
            pl.BlockSpec((1, H_E, 4 * H_E), lambda d, i: (d, 0, 0)),
            pl.BlockSpec((1, 1, 4 * H_E), lambda d, i: (d, 0, 0)),
        ],
        out_specs=pl.BlockSpec((BT, N_B, H_E),
                               lambda d, i: (i + d * (G_T - 1 - 2 * i), 0, d)),
        out_shape=jax.ShapeDtypeStruct((L_S, N_B, 2 * H_E), jnp.float32),
        scratch_shapes=[
            pltpu.VMEM((2, BT * N_B, E_D), jnp.float32),
            pltpu.VMEM((BT * N_B, 4 * H_E), jnp.float32),
            pltpu.VMEM((N_B, H_E), jnp.float32),
            pltpu.VMEM((N_B, H_E), jnp.float32),
            pltpu.SemaphoreType.DMA,
        ],
        compiler_params=pltpu.CompilerParams(
            dimension_semantics=("arbitrary", "arbitrary"),
            vmem_limit_bytes=56 * 1024 * 1024),
        name="enc0_fused",
    )(ids_flat, embed, wx, wh, b)


def _sigm(x):
    return jax.nn.sigmoid(x)


def _make_lstm_body(din):
    def body(x_ref, wx_ref, wh_ref, b_ref, out_ref, gin_ref, h_ref, c_ref):
        d = pl.program_id(0)
        i = pl.program_id(1)

        @pl.when(i == 0)
        def _():
            h_ref[...] = jnp.zeros((N_B, H_E), jnp.float32)
            c_ref[...] = jnp.zeros((N_B, H_E), jnp.float32)

        gin_ref[...] = jnp.dot(x_ref[...], wx_ref[0],
                               preferred_element_type=jnp.float32)

        for j in range(BT):
            row = j + d * (BT - 1 - 2 * j)
            base = pl.multiple_of(row * N_B, N_B)
            g = (gin_ref[pl.ds(base, N_B), :] + jnp.dot(
                h_ref[...], wh_ref[0],
                preferred_element_type=jnp.float32)) + b_ref[0]
            gi = _sigm(g[:, :H_E])
            gf = _sigm(g[:, H_E:2 * H_E])
            gg = jnp.tanh(g[:, 2 * H_E:3 * H_E])
            go = _sigm(g[:, 3 * H_E:])
            c = gf * c_ref[...] + gi * gg
            c_ref[...] = c
            h = go * jnp.tanh(c)
            h_ref[...] = h
            out_ref[row] = h

    return body


def _lstm_layer(xs, wx, wh, b, din):
    return pl.pallas_call(
        _make_lstm_body(din),
        grid=(2, G_T),
        in_specs=[
            pl.BlockSpec((BT * N_B, din),
                         lambda d, i: (i + d * (G_T - 1 - 2 * i), 0)),
            pl.BlockSpec((1, din, 4 * H_E), lambda d, i: (d, 0, 0)),
            pl.BlockSpec((1, H_E, 4 * H_E), lambda d, i: (d, 0, 0)),
            pl.BlockSpec((1, 1, 4 * H_E), lambda d, i: (d, 0, 0)),
        ],
        out_specs=pl.BlockSpec((BT, N_B, H_E),
                               lambda d, i: (i + d * (G_T - 1 - 2 * i), 0, d)),
        out_shape=jax.ShapeDtypeStruct((L_S, N_B, 2 * H_E), jnp.float32),
        scratch_shapes=[
            pltpu.VMEM((BT * N_B, 4 * H_E), jnp.float32),
            pltpu.VMEM((N_B, H_E), jnp.float32),
            pltpu.VMEM((N_B, H_E), jnp.float32),
        ],
        compiler_params=pltpu.CompilerParams(
            dimension_semantics=("arbitrary", "arbitrary"),
            vmem_limit_bytes=56 * 1024 * 1024),
        name=f"bilstm_{din}",
    )(xs, wx, wh, b)


def _dec_body(enc_ref, tags_ref, wa_ref, wb_ref, b_ref, ow_ref, ob_ref,
              probs_ref, lacc_ref, inp_ref, h_ref, c_ref):
    i = pl.program_id(0)
    lanes = jax.lax.broadcasted_iota(jnp.int32, (N_B, T_T), 1)

    @pl.when(i == 0)
    def _():
        inp_ref[...] = jnp.zeros_like(inp_ref)
        h_ref[...] = jnp.zeros_like(h_ref)
        bwd0 = enc_ref[0][:, H_E:]
        c_ref[...] = jnp.concatenate([bwd0, bwd0], axis=-1)
        lacc_ref[...] = jnp.zeros_like(lacc_ref)

    for j in range(BT):
        enc_t = enc_ref[j]
        h_in = h_ref[...] + enc_t
        g = (jnp.dot(inp_ref[...], wa_ref[...],
                     preferred_element_type=jnp.float32) +
             jnp.dot(h_in, wb_ref[...],
                     preferred_element_type=jnp.float32)) + b_ref[...]
        gi = _sigm(g[:, :H_D])
        gf = _sigm(g[:, H_D:2 * H_D])
        gg = jnp.tanh(g[:, 2 * H_D:3 * H_D])
        go = _sigm(g[:, 3 * H_D:])
        c = gf * c_ref[...] + gi * gg
        c_ref[...] = c
        h = go * jnp.tanh(c)
        h_ref[...] = h
        logits = jnp.dot(h, ow_ref[...],
                         preferred_element_type=jnp.float32) + ob_ref[...]
        m = jnp.max(logits, axis=-1, keepdims=True)
        xm = logits - m
        e = jnp.exp(xm)
        s = jnp.sum(e, axis=-1, keepdims=True)
        logp = xm - jnp.log(s)
        prob = jnp.exp(logp)
        probs_ref[:, j, :] = prob
        tag = tags_ref[j]
        onehot = lanes == tag
        lacc_ref[...] += jnp.where(onehot, logp, 0.0)
        pm = jnp.max(prob, axis=-1, keepdims=True)
        first = jnp.min(jnp.where(prob == pm, lanes, T_T),
                        axis=-1, keepdims=True)
        inp_ref[...] = (lanes == first).astype(jnp.float32)


def _decoder(enc3, tags3, wa, wb, bd, ow, ob):
    return pl.pallas_call(
        _dec_body,
        grid=(G_T,),
        in_specs=[
            pl.BlockSpec((BT, N_B, H_D), lambda i: (i, 0, 0)),
            pl.BlockSpec((BT, N_B, 1), lambda i: (i, 0, 0)),
            pl.BlockSpec((T_T, 4 * H_D), lambda i: (0, 0)),
            pl.BlockSpec((H_D, 4 * H_D), lambda i: (0, 0)),
            pl.BlockSpec((1, 4 * H_D), lambda i: (0, 0)),
            pl.BlockSpec((H_D, T_T), lambda i: (0, 0)),
            pl.BlockSpec((1, T_T), lambda i: (0, 0)),
        ],
        out_specs=[
            pl.BlockSpec((N_B, BT, T_T), lambda i: (0, i, 0)),
            pl.BlockSpec((N_B, T_T), lambda i: (0, 0)),
        ],
        out_shape=[
            jax.ShapeDtypeStruct((N_B, L_S, T_T), jnp.float32),
            jax.ShapeDtypeStruct((N_B, T_T), jnp.float32),
        ],
        scratch_shapes=[
            pltpu.VMEM((N_B, T_T), jnp.float32),
            pltpu.VMEM((N_B, H_D), jnp.float32),
            pltpu.VMEM((N_B, H_D), jnp.float32),
        ],
        compiler_params=pltpu.CompilerParams(
            dimension_semantics=("arbitrary",),
            vmem_limit_bytes=56 * 1024 * 1024),
        name="decoder",
    )(enc3, tags3, wa, wb, bd, ow, ob)


def kernel(input_ids, tag_ids, embed, enc0_wih, enc0_whh, enc0_b,
           enc1_wih, enc1_whh, enc1_b, dec_wih, dec_whh, dec_b,
           out_w, out_b):
    ids_flat = input_ids.T.reshape(L_S * N_B).astype(jnp.int32)
    h0 = _enc0_layer(ids_flat, embed,
                     jnp.transpose(enc0_wih, (0, 2, 1)),
                     jnp.transpose(enc0_whh, (0, 2, 1)),
                     enc0_b[:, None, :])

    enc = _lstm_layer(h0.reshape(L_S * N_B, 2 * H_E),
                      jnp.transpose(enc1_wih, (0, 2, 1)),
                      jnp.transpose(enc1_whh, (0, 2, 1)),
                      enc1_b[:, None, :], 2 * H_E)

    tags3 = tag_ids.T[..., None]
    prob, lacc = _decoder(enc, tags3.astype(jnp.int32), dec_wih.T, dec_whh.T,
                          dec_b[None, :], out_w.T, out_b[None, :])
    loss = -jnp.sum(lacc) / N_B
    return prob, loss


# speedup vs baseline: 1.1104x; 1.1104x over previous
import jax
import jax.numpy as jnp
from jax.experimental import pallas as pl
from jax.experimental.pallas import tpu as pltpu

N_B, L_S = 128, 160
V_SZ, E_D = 30000, 512
H_E = 512
H_D = 1024
T_T = 128
BT = 8
G_T = L_S // BT
NH = N_B // 2


def _gather_body(ids_ref, embed_ref, x_ref, sem):
    d = pl.program_id(0)
    i = pl.program_id(1)
    base = (d * (G_T // 2) + i) * (BT * N_B)

    def issue8(k8, carry):
        k = k8 * 8
        for u in range(8):
            tok = ids_ref[base + k + u]
            pltpu.make_async_copy(embed_ref.at[tok], x_ref.at[k + u],
                                  sem).start()
        return carry

    jax.lax.fori_loop(0, (BT * N_B) // 8, issue8, 0)
    pltpu.make_async_copy(x_ref, x_ref, sem).wait()


def _embed_gather(ids_flat, embed):
    return pl.pallas_call(
        _gather_body,
        grid=(2, G_T // 2),
        in_specs=[
            pl.BlockSpec(memory_space=pltpu.SMEM),
            pl.BlockSpec(memory_space=pl.ANY),
        ],
        out_specs=pl.BlockSpec((BT * N_B, E_D),
                               lambda d, i: (d * (G_T // 2) + i, 0)),
        out_shape=jax.ShapeDtypeStruct((L_S * N_B, E_D), jnp.float32),
        scratch_shapes=[pltpu.SemaphoreType.DMA],
        compiler_params=pltpu.CompilerParams(
            dimension_semantics=("arbitrary", "arbitrary")),
        name="embed_gather",
    )(ids_flat, embed)


def _sigm(x):
    return jax.nn.sigmoid(x)


def _make_lstm_body(din):
    def body(x_ref, wx_ref, wh_ref, b_ref, out_ref, gin_ref, h_ref, c_ref):
        d = pl.program_id(0)
        i = pl.program_id(1)

        @pl.when(i == 0)
        def _():
            h_ref[...] = jnp.zeros((N_B, H_E), jnp.float32)
            c_ref[...] = jnp.zeros((N_B, H_E), jnp.float32)

        gin_ref[...] = jnp.dot(x_ref[...], wx_ref[0],
                               preferred_element_type=jnp.float32)

        for j in range(BT):
            row = j + d * (BT - 1 - 2 * j)
            base = pl.multiple_of(row * N_B, N_B)
            g = (gin_ref[pl.ds(base, N_B), :] + jnp.dot(
                h_ref[...], wh_ref[0],
                preferred_element_type=jnp.float32)) + b_ref[0]
            gi = _sigm(g[:, :H_E])
            gf = _sigm(g[:, H_E:2 * H_E])
            gg = jnp.tanh(g[:, 2 * H_E:3 * H_E])
            go = _sigm(g[:, 3 * H_E:])
            c = gf * c_ref[...] + gi * gg
            c_ref[...] = c
            h = go * jnp.tanh(c)
            h_ref[...] = h
            out_ref[row] = h

    return body


def _lstm_layer(xs, wx, wh, b, din):
    return pl.pallas_call(
        _make_lstm_body(din),
        grid=(2, G_T),
        in_specs=[
            pl.BlockSpec((BT * N_B, din),
                         lambda d, i: (i + d * (G_T - 1 - 2 * i), 0)),
            pl.BlockSpec((1, din, 4 * H_E), lambda d, i: (d, 0, 0)),
            pl.BlockSpec((1, H_E, 4 * H_E), lambda d, i: (d, 0, 0)),
            pl.BlockSpec((1, 1, 4 * H_E), lambda d, i: (d, 0, 0)),
        ],
        out_specs=pl.BlockSpec((BT, N_B, H_E),
                               lambda d, i: (i + d * (G_T - 1 - 2 * i), 0, d)),
        out_shape=jax.ShapeDtypeStruct((L_S, N_B, 2 * H_E), jnp.float32),
        scratch_shapes=[
            pltpu.VMEM((BT * N_B, 4 * H_E), jnp.float32),
            pltpu.VMEM((N_B, H_E), jnp.float32),
            pltpu.VMEM((N_B, H_E), jnp.float32),
        ],
        compiler_params=pltpu.CompilerParams(
            dimension_semantics=("arbitrary", "arbitrary"),
            vmem_limit_bytes=56 * 1024 * 1024),
        name=f"bilstm_{din}",
    )(xs, wx, wh, b)


def _dec_body(enc_ref, tags_ref, wa_ref, wb_ref, b_ref, ow_ref, ob_ref,
              probs_ref, lacc_ref, inp_ref, h_ref, c_ref):
    i = pl.program_id(0)
    lanes = jax.lax.broadcasted_iota(jnp.int32, (N_B, T_T), 1)

    @pl.when(i == 0)
    def _():
        inp_ref[...] = jnp.zeros_like(inp_ref)
        h_ref[...] = jnp.zeros_like(h_ref)
        bwd0 = enc_ref[0][:, H_E:]
        c_ref[...] = jnp.concatenate([bwd0, bwd0], axis=-1)
        lacc_ref[...] = jnp.zeros_like(lacc_ref)

    for j in range(BT):
        enc_t = enc_ref[j]
        h_in = h_ref[...] + enc_t
        g = (jnp.dot(inp_ref[...], wa_ref[...],
                     preferred_element_type=jnp.float32) +
             jnp.dot(h_in, wb_ref[...],
                     preferred_element_type=jnp.float32)) + b_ref[...]
        gi = _sigm(g[:, :H_D])
        gf = _sigm(g[:, H_D:2 * H_D])
        gg = jnp.tanh(g[:, 2 * H_D:3 * H_D])
        go = _sigm(g[:, 3 * H_D:])
        c = gf * c_ref[...] + gi * gg
        c_ref[...] = c
        h = go * jnp.tanh(c)
        h_ref[...] = h
        logits = jnp.dot(h, ow_ref[...],
                         preferred_element_type=jnp.float32) + ob_ref[...]
        m = jnp.max(logits, axis=-1, keepdims=True)
        xm = logits - m
        e = jnp.exp(xm)
        s = jnp.sum(e, axis=-1, keepdims=True)
        logp = xm - jnp.log(s)
        prob = jnp.exp(logp)
        probs_ref[:, j, :] = prob
        tag = tags_ref[j]
        onehot = lanes == tag
        lacc_ref[...] += jnp.where(onehot, logp, 0.0)
        pm = jnp.max(prob, axis=-1, keepdims=True)
        first = jnp.min(jnp.where(prob == pm, lanes, T_T),
                        axis=-1, keepdims=True)
        inp_ref[...] = (lanes == first).astype(jnp.float32)


def _decoder(enc3, tags3, wa, wb, bd, ow, ob):
    return pl.pallas_call(
        _dec_body,
        grid=(G_T,),
        in_specs=[
            pl.BlockSpec((BT, N_B, H_D), lambda i: (i, 0, 0)),
            pl.BlockSpec((BT, N_B, 1), lambda i: (i, 0, 0)),
            pl.BlockSpec((T_T, 4 * H_D), lambda i: (0, 0)),
            pl.BlockSpec((H_D, 4 * H_D), lambda i: (0, 0)),
            pl.BlockSpec((1, 4 * H_D), lambda i: (0, 0)),
            pl.BlockSpec((H_D, T_T), lambda i: (0, 0)),
            pl.BlockSpec((1, T_T), lambda i: (0, 0)),
        ],
        out_specs=[
            pl.BlockSpec((N_B, BT, T_T), lambda i: (0, i, 0)),
            pl.BlockSpec((N_B, T_T), lambda i: (0, 0)),
        ],
        out_shape=[
            jax.ShapeDtypeStruct((N_B, L_S, T_T), jnp.float32),
            jax.ShapeDtypeStruct((N_B, T_T), jnp.float32),
        ],
        scratch_shapes=[
            pltpu.VMEM((N_B, T_T), jnp.float32),
            pltpu.VMEM((N_B, H_D), jnp.float32),
            pltpu.VMEM((N_B, H_D), jnp.float32),
        ],
        compiler_params=pltpu.CompilerParams(
            dimension_semantics=("arbitrary",),
            vmem_limit_bytes=56 * 1024 * 1024),
        name="decoder",
    )(enc3, tags3, wa, wb, bd, ow, ob)


def kernel(input_ids, tag_ids, embed, enc0_wih, enc0_whh, enc0_b,
           enc1_wih, enc1_whh, enc1_b, dec_wih, dec_whh, dec_b,
           out_w, out_b):
    ids_flat = input_ids.T.reshape(L_S * N_B).astype(jnp.int32)
    x = _embed_gather(ids_flat, embed)
    h0 = _lstm_layer(x,
                     jnp.transpose(enc0_wih, (0, 2, 1)),
                     jnp.transpose(enc0_whh, (0, 2, 1)),
                     enc0_b[:, None, :], E_D)

    enc = _lstm_layer(h0.reshape(L_S * N_B, 2 * H_E),
                      jnp.transpose(enc1_wih, (0, 2, 1)),
                      jnp.transpose(enc1_whh, (0, 2, 1)),
                      enc1_b[:, None, :], 2 * H_E)

    tags3 = tag_ids.T[..., None]
    prob, lacc = _decoder(enc, tags3.astype(jnp.int32), dec_wih.T, dec_whh.T,
                          dec_b[None, :], out_w.T, out_b[None, :])
    loss = -jnp.sum(lacc) / N_B
    return prob, loss
